# baseline (speedup 1.0000x reference)
import functools
import math

import jax
import jax.numpy as jnp
from jax import lax
from jax.experimental import pallas as pl
from jax.experimental.pallas import tpu as pltpu
from jax.experimental.pallas import tpu_sc as plsc

DIM = 1024
SCALE = math.sqrt(DIM)
LANES = 16

NUM_WORKERS = 32
NBUF = 8
CHUNK = 8
AHEAD = 6
RUNROLL = 2


def kernel(x, table):
    batch, seq = x.shape
    n = batch * seq
    rows_per_worker = n // NUM_WORKERS
    nchunks = rows_per_worker // CHUNK
    assert nchunks % NBUF == 0 and CHUNK % RUNROLL == 0

    idx = x.reshape(n).astype(jnp.int32)
    mesh = plsc.VectorSubcoreMesh(
        core_axis_name="core", subcore_axis_name="subcore"
    )

    @functools.partial(
        pl.kernel,
        out_type=jax.ShapeDtypeStruct((n, DIM), jnp.float32),
        mesh=mesh,
        scratch_types=(
            [
                pltpu.VMEM((rows_per_worker,), jnp.int32),
                pltpu.VMEM((NBUF, CHUNK, DIM), jnp.float32),
            ]
            + [pltpu.SemaphoreType.DMA] * (2 * NBUF)
        ),
    )
    def emb_kernel(table_hbm, idx_hbm, out_hbm, idx_v, bufs, *sems):
        gsem = sems[:NBUF]
        ssem = sems[NBUF:]
        wid = lax.axis_index("subcore") * 2 + lax.axis_index("core")
        base = wid * rows_per_worker

        pltpu.sync_copy(idx_hbm.at[pl.ds(base, rows_per_worker)], idx_v)

        def gather_desc(j, b):
            return pltpu.make_async_copy(
                table_hbm.at[idx_v.at[pl.ds(j * CHUNK, CHUNK)]],
                bufs.at[b],
                gsem[b],
            )

        def scatter_desc(j, b):
            return pltpu.make_async_copy(
                bufs.at[b],
                out_hbm.at[pl.ds(base + j * CHUNK, CHUNK)],
                ssem[b],
            )

        for j0 in range(AHEAD):
            gather_desc(j0, j0).start()

        @pl.loop(0, nchunks, step=NBUF)
        def _(g):
            for b in range(NBUF):
                j = g + b
                k = j + AHEAD
                bk = (b + AHEAD) % NBUF

                @pl.when(jnp.logical_and(k >= NBUF, k < nchunks))
                def _():
                    scatter_desc(k - NBUF, bk).wait()

                @pl.when(k < nchunks)
                def _():
                    gather_desc(k, bk).start()

                gather_desc(j, b).wait()

                @pl.loop(0, CHUNK, step=RUNROLL)
                def _(r):
                    for dr in range(RUNROLL):
                        for v in range(DIM // LANES):
                            sl = pl.ds(v * LANES, LANES)
                            bufs.at[b, r + dr, sl][...] = (
                                bufs.at[b, r + dr, sl][...] * SCALE
                            )

                scatter_desc(j, b).start()

        for b in range(NBUF):
            scatter_desc(nchunks - NBUF + b, b).wait()

    out = emb_kernel(table, idx)
    return out.reshape(batch, seq, DIM)
